# baseline (speedup 1.0000x reference)
import functools

import jax
import jax.numpy as jnp
import numpy as np
from jax import lax
from jax.experimental import pallas as pl
from jax.experimental.pallas import tpu as pltpu
from jax.experimental.pallas import tpu_sc as plsc

_VOCAB = 500000
_DIM = 64
_PRIME = 31

_B = 8
_T = 512
_NW = 32
_CHUNK = (_B * _T) // _NW
_CHUNKS_PER_ROW = _T // _CHUNK
_PAD = 8
_L = 16

_COLS = np.empty((_DIM,), np.int32)
for _d in range(_DIM):
    _g, _r = divmod(_d, 2 * _L)
    _COLS[_d] = _g * 2 * _L + (_r // 2) + _L * (_r % 2)


def _body(byte_hbm, emb2_hbm, emb3_hbm, emb4_hbm, out_hbm,
          bytes_v, idx2_v, idx3_v, idx4_v, off2_v, off3_v, off4_v,
          g2_v, g3_v, g4_v, o_v, sem):
    nc = 2
    wid = lax.axis_index("s") * nc + lax.axis_index("c")
    b = wid // _CHUNKS_PER_ROW
    p0 = (wid % _CHUNKS_PER_ROW) * _CHUNK

    bytes_v[pl.ds(0, _L)] = jnp.zeros((_L,), jnp.int32)
    pltpu.sync_copy(byte_hbm, bytes_v.at[pl.ds(_PAD, _B * _T)])

    base = b * _T + p0
    for g in range(_CHUNK // _L):
        t0 = base + g * _L
        v0 = bytes_v[pl.ds(_PAD + t0, _L)]
        v1 = bytes_v[pl.ds(_PAD + t0 - 1, _L)]
        v2 = bytes_v[pl.ds(_PAD + t0 - 2, _L)]
        v3 = bytes_v[pl.ds(_PAD + t0 - 3, _L)]
        h2 = v1 * _PRIME + v0
        h3 = v2 * (_PRIME * _PRIME) + h2
        h4 = lax.rem(v3 * (_PRIME * _PRIME * _PRIME) + h3, _VOCAB)
        sl = pl.ds(g * _L, _L)
        idx2_v[sl] = lax.shift_right_logical(h2, 2)
        idx3_v[sl] = lax.shift_right_logical(h3, 2)
        idx4_v[sl] = lax.shift_right_logical(h4, 2)
        off2_v[sl] = (h2 & 3) * (2 * _L)
        off3_v[sl] = (h3 & 3) * (2 * _L)
        off4_v[sl] = (h4 & 3) * (2 * _L)

    c2 = pltpu.async_copy(emb2_hbm.at[idx2_v], g2_v, sem)
    c3 = pltpu.async_copy(emb3_hbm.at[idx3_v], g3_v, sem)
    c4 = pltpu.async_copy(emb4_hbm.at[idx4_v], g4_v, sem)
    c2.wait()
    c3.wait()
    c4.wait()

    def pos_row(t, g_v, off_v, quarter):
        q32 = off_v[pl.ds(t, _L)][0]
        chunks = []
        for cc in range(2):
            w = g_v[t, pl.ds(q32 + cc * _L, _L)]
            a = plsc.bitcast(lax.shift_left(w, 16), jnp.float32)
            bb = plsc.bitcast(w & jnp.int32(-65536), jnp.float32)
            chunks += [a * quarter, bb * quarter]
        return chunks

    def acc_body(t, _):
        r2 = pos_row(t, g2_v, off2_v, 0.25)
        r3 = pos_row(t, g3_v, off3_v, 0.25)
        r4 = pos_row(t, g4_v, off4_v, 0.25)
        for c in range(_DIM // _L):
            o_v[t, pl.ds(c * _L, _L)] = r2[c] + r3[c] + r4[c]
        return 0

    lax.fori_loop(0, _CHUNK, acc_body, 0)

    @pl.when(p0 == 0)
    def _fixup():
        r2a = pos_row(1, g2_v, off2_v, 0.25)
        r2b = pos_row(2, g2_v, off2_v, 0.25)
        r3b = pos_row(2, g3_v, off3_v, 0.25)
        for c in range(_DIM // _L):
            sl = pl.ds(c * _L, _L)
            o_v[0, sl] = jnp.zeros((_L,), jnp.float32)
            o_v[1, sl] = r2a[c]
            o_v[2, sl] = r2b[c] + r3b[c]

    pltpu.sync_copy(o_v, out_hbm.at[b, pl.ds(p0, _CHUNK), :])


@jax.jit
def kernel(byte_ids, emb_2, emb_3, emb_4):
    mesh = plsc.VectorSubcoreMesh(core_axis_name="c", subcore_axis_name="s")
    f = functools.partial(
        pl.kernel,
        mesh=mesh,
        compiler_params=pltpu.CompilerParams(
            use_tc_tiling_on_sc=True, needs_layout_passes=False),
        out_type=jax.ShapeDtypeStruct((_B, _T, 2 * _DIM), jnp.float32),
        scratch_types=[
            pltpu.VMEM((_PAD + _B * _T,), jnp.int32),
            pltpu.VMEM((_CHUNK,), jnp.int32),
            pltpu.VMEM((_CHUNK,), jnp.int32),
            pltpu.VMEM((_CHUNK,), jnp.int32),
            pltpu.VMEM((_CHUNK + _L,), jnp.int32),
            pltpu.VMEM((_CHUNK + _L,), jnp.int32),
            pltpu.VMEM((_CHUNK + _L,), jnp.int32),
            pltpu.VMEM((_CHUNK, 2 * _DIM), jnp.int32),
            pltpu.VMEM((_CHUNK, 2 * _DIM), jnp.int32),
            pltpu.VMEM((_CHUNK, 2 * _DIM), jnp.int32),
            pltpu.VMEM((_CHUNK, 2 * _DIM), jnp.float32),
            pltpu.SemaphoreType.DMA,
        ],
    )(_body)
    def prep(e):
        b32 = jax.lax.bitcast_convert_type(e, jnp.uint32)
        lo = b32[:, 0::2]
        hi = b32[:, 1::2]
        r_lo = (lo + 0x7FFF + ((lo >> 16) & 1)) >> 16
        r_hi = (hi + 0x7FFF + ((hi >> 16) & 1)) & jnp.uint32(0xFFFF0000)
        return jax.lax.bitcast_convert_type(r_hi | r_lo, jnp.int32).reshape(
            _VOCAB // 4, 2 * _DIM)

    out_pad = f(byte_ids.reshape(-1), prep(emb_2), prep(emb_3), prep(emb_4))
    return out_pad[:, :, jnp.asarray(_COLS)]


# speedup vs baseline: 2.7537x; 2.7537x over previous
import functools

import jax
import jax.numpy as jnp
import numpy as np
from jax import lax
from jax.experimental import pallas as pl
from jax.experimental.pallas import tpu as pltpu
from jax.experimental.pallas import tpu_sc as plsc

_VOCAB = 500000
_DIM = 64
_PRIME = 31

_B = 8
_T = 512
_NW = 32
_CHUNK = (_B * _T) // _NW
_CHUNKS_PER_ROW = _T // _CHUNK
_PAD = 8
_L = 16


def _body(byte_hbm, emb2_hbm, emb3_hbm, emb4_hbm, out_hbm,
          bytes_v, idx2_v, idx3_v, idx4_v, off2_v, off3_v, off4_v,
          g2_v, g3_v, g4_v, o_v, sem):
    nc = 2
    wid = lax.axis_index("s") * nc + lax.axis_index("c")
    b = wid // _CHUNKS_PER_ROW
    p0 = (wid % _CHUNKS_PER_ROW) * _CHUNK

    bytes_v[pl.ds(0, _L)] = jnp.zeros((_L,), jnp.int32)
    pltpu.sync_copy(byte_hbm, bytes_v.at[pl.ds(_PAD, _B * _T)])

    base = b * _T + p0
    for g in range(_CHUNK // _L):
        t0 = base + g * _L
        v0 = bytes_v[pl.ds(_PAD + t0, _L)]
        v1 = bytes_v[pl.ds(_PAD + t0 - 1, _L)]
        v2 = bytes_v[pl.ds(_PAD + t0 - 2, _L)]
        v3 = bytes_v[pl.ds(_PAD + t0 - 3, _L)]
        h2 = v1 * _PRIME + v0
        h3 = v2 * (_PRIME * _PRIME) + h2
        h4 = lax.rem(v3 * (_PRIME * _PRIME * _PRIME) + h3, _VOCAB)
        sl = pl.ds(g * _L, _L)
        idx2_v[sl] = lax.shift_right_logical(h2, 2)
        idx3_v[sl] = lax.shift_right_logical(h3, 2)
        idx4_v[sl] = lax.shift_right_logical(h4, 2)
        off2_v[sl] = (h2 & 3) * (2 * _L)
        off3_v[sl] = (h3 & 3) * (2 * _L)
        off4_v[sl] = (h4 & 3) * (2 * _L)

    c2 = pltpu.async_copy(emb2_hbm.at[idx2_v], g2_v, sem)
    c3 = pltpu.async_copy(emb3_hbm.at[idx3_v], g3_v, sem)
    c4 = pltpu.async_copy(emb4_hbm.at[idx4_v], g4_v, sem)
    c2.wait()
    c3.wait()
    c4.wait()

    def pos_row(t, g_v, off_v, quarter):
        q32 = off_v[pl.ds(t, _L)][0]
        los, his = [], []
        for cc in range(2):
            w = g_v[t, pl.ds(q32 + cc * _L, _L)]
            los.append(plsc.bitcast(lax.shift_left(w, 16), jnp.float32)
                       * quarter)
            his.append(plsc.bitcast(w & jnp.int32(-65536), jnp.float32)
                       * quarter)
        return los + his

    def acc_body(t, _):
        r2 = pos_row(t, g2_v, off2_v, 0.25)
        r3 = pos_row(t, g3_v, off3_v, 0.25)
        r4 = pos_row(t, g4_v, off4_v, 0.25)
        for c in range(_DIM // _L):
            o_v[t, pl.ds(c * _L, _L)] = r2[c] + r3[c] + r4[c]
        return 0

    lax.fori_loop(0, _CHUNK, acc_body, 0)

    @pl.when(p0 == 0)
    def _fixup():
        r2a = pos_row(1, g2_v, off2_v, 0.25)
        r2b = pos_row(2, g2_v, off2_v, 0.25)
        r3b = pos_row(2, g3_v, off3_v, 0.25)
        for c in range(_DIM // _L):
            sl = pl.ds(c * _L, _L)
            o_v[0, sl] = jnp.zeros((_L,), jnp.float32)
            o_v[1, sl] = r2a[c]
            o_v[2, sl] = r2b[c] + r3b[c]

    pltpu.sync_copy(o_v, out_hbm.at[b, pl.ds(p0, _CHUNK), :])


@jax.jit
def kernel(byte_ids, emb_2, emb_3, emb_4):
    mesh = plsc.VectorSubcoreMesh(core_axis_name="c", subcore_axis_name="s")
    f = functools.partial(
        pl.kernel,
        mesh=mesh,
        compiler_params=pltpu.CompilerParams(
            use_tc_tiling_on_sc=True, needs_layout_passes=False),
        out_type=jax.ShapeDtypeStruct((_B, _T, 2 * _DIM), jnp.float32),
        scratch_types=[
            pltpu.VMEM((_PAD + _B * _T,), jnp.int32),
            pltpu.VMEM((_CHUNK,), jnp.int32),
            pltpu.VMEM((_CHUNK,), jnp.int32),
            pltpu.VMEM((_CHUNK,), jnp.int32),
            pltpu.VMEM((_CHUNK + _L,), jnp.int32),
            pltpu.VMEM((_CHUNK + _L,), jnp.int32),
            pltpu.VMEM((_CHUNK + _L,), jnp.int32),
            pltpu.VMEM((_CHUNK, 2 * _DIM), jnp.int32),
            pltpu.VMEM((_CHUNK, 2 * _DIM), jnp.int32),
            pltpu.VMEM((_CHUNK, 2 * _DIM), jnp.int32),
            pltpu.VMEM((_CHUNK, 2 * _DIM), jnp.float32),
            pltpu.SemaphoreType.DMA,
        ],
    )(_body)
    def prep(e):
        b32 = jax.lax.bitcast_convert_type(e, jnp.uint32)
        lo = b32[:, : _DIM // 2]
        hi = b32[:, _DIM // 2:]
        r_lo = (lo + 0x7FFF + ((lo >> 16) & 1)) >> 16
        r_hi = (hi + 0x7FFF + ((hi >> 16) & 1)) & jnp.uint32(0xFFFF0000)
        return jax.lax.bitcast_convert_type(r_hi | r_lo, jnp.int32).reshape(
            _VOCAB // 4, 2 * _DIM)

    out_pad = f(byte_ids.reshape(-1), prep(emb_2), prep(emb_3), prep(emb_4))
    return out_pad[:, :, :_DIM]


# speedup vs baseline: 3.5753x; 1.2984x over previous
import functools

import jax
import jax.numpy as jnp
from jax import lax
from jax.experimental import pallas as pl
from jax.experimental.pallas import tpu as pltpu
from jax.experimental.pallas import tpu_sc as plsc

_VOCAB = 500000
_DIM = 64
_PRIME = 31

_B = 8
_T = 512
_NW = 32
_CHUNK = (_B * _T) // _NW
_CHUNKS_PER_ROW = _T // _CHUNK
_PAD = 8
_L = 16


def _body(byte_hbm, ct23_hbm, ct42_hbm, out_hbm,
          bytes_v, idx2_v, idx3_v, idx4_v, g2_v, g3_v, g4_v, o_v, sem):
    nc = 2
    wid = lax.axis_index("s") * nc + lax.axis_index("c")
    b = wid // _CHUNKS_PER_ROW
    p0 = (wid % _CHUNKS_PER_ROW) * _CHUNK

    bytes_v[pl.ds(0, _L)] = jnp.zeros((_L,), jnp.int32)
    pltpu.sync_copy(byte_hbm, bytes_v.at[pl.ds(_PAD, _B * _T)])

    base = b * _T + p0
    for g in range(_CHUNK // _L):
        t0 = base + g * _L
        v0 = bytes_v[pl.ds(_PAD + t0, _L)]
        v1 = bytes_v[pl.ds(_PAD + t0 - 1, _L)]
        v2 = bytes_v[pl.ds(_PAD + t0 - 2, _L)]
        v3 = bytes_v[pl.ds(_PAD + t0 - 3, _L)]
        h2 = v1 * _PRIME + v0
        h3 = v2 * (_PRIME * _PRIME) + h2
        h4 = lax.rem(v3 * (_PRIME * _PRIME * _PRIME) + h3, _VOCAB)
        sl = pl.ds(g * _L, _L)
        idx2_v[sl] = h2
        idx3_v[sl] = h3
        idx4_v[sl] = h4

    c2 = pltpu.async_copy(ct23_hbm.at[idx2_v], g2_v, sem)
    c3 = pltpu.async_copy(ct23_hbm.at[idx3_v], g3_v, sem)
    c4 = pltpu.async_copy(ct42_hbm.at[idx4_v], g4_v, sem)
    c2.wait()
    c3.wait()
    c4.wait()

    def acc_body(t, _):
        for c in range(_DIM // _L):
            sl = pl.ds(c * _L, _L)
            sl3 = pl.ds(_DIM + c * _L, _L)
            o_v[t, sl] = (g2_v[t, sl] + g3_v[t, sl3] + g4_v[t, sl]) * 0.25
        return 0

    lax.fori_loop(0, _CHUNK, acc_body, 0)

    @pl.when(p0 == 0)
    def _fixup():
        for c in range(_DIM // _L):
            sl = pl.ds(c * _L, _L)
            o_v[0, sl] = jnp.zeros((_L,), jnp.float32)
            o_v[1, sl] = g2_v[1, sl] * 0.25
            o_v[2, sl] = (g2_v[2, sl] + g3_v[2, pl.ds(_DIM + c * _L, _L)]) * 0.25

    pltpu.sync_copy(o_v, out_hbm.at[b, pl.ds(p0, _CHUNK), :])


@jax.jit
def kernel(byte_ids, emb_2, emb_3, emb_4):
    mesh = plsc.VectorSubcoreMesh(core_axis_name="c", subcore_axis_name="s")
    f = functools.partial(
        pl.kernel,
        mesh=mesh,
        compiler_params=pltpu.CompilerParams(use_tc_tiling_on_sc=True),
        out_type=jax.ShapeDtypeStruct((_B, _T, 2 * _DIM), jnp.float32),
        scratch_types=[
            pltpu.VMEM((_PAD + _B * _T,), jnp.int32),
            pltpu.VMEM((_CHUNK,), jnp.int32),
            pltpu.VMEM((_CHUNK,), jnp.int32),
            pltpu.VMEM((_CHUNK,), jnp.int32),
            pltpu.VMEM((_CHUNK, 2 * _DIM), jnp.float32),
            pltpu.VMEM((_CHUNK, 2 * _DIM), jnp.float32),
            pltpu.VMEM((_CHUNK, 2 * _DIM), jnp.float32),
            pltpu.VMEM((_CHUNK, 2 * _DIM), jnp.float32),
            pltpu.SemaphoreType.DMA,
        ],
    )(_body)

    ct23 = jnp.concatenate([emb_2, emb_3], axis=1)
    ct42 = jnp.concatenate([emb_4, emb_2], axis=1)
    out_pad = f(byte_ids.reshape(-1), ct23, ct42)
    return out_pad[:, :, :_DIM]


# speedup vs baseline: 3.6079x; 1.0091x over previous
import functools

import jax
import jax.numpy as jnp
from jax import lax
from jax.experimental import pallas as pl
from jax.experimental.pallas import tpu as pltpu
from jax.experimental.pallas import tpu_sc as plsc

_NGRAM_SIZES = (2, 3, 4)
_VOCAB = 500000
_DIM = 64
_PRIME = 31

_B = 8
_T = 512
_NW = 32
_CHUNK = (_B * _T) // _NW
_CHUNKS_PER_ROW = _T // _CHUNK
_PAD = 8
_L = 16


def _body(byte_hbm, emb2_hbm, emb3_hbm, emb4_hbm, out_hbm,
          bytes_v, idx2_v, idx3_v, idx4_v, e2_v, e3_v, e4_v, out_v, sem):
    nc = 2
    wid = lax.axis_index("s") * nc + lax.axis_index("c")
    b = wid // _CHUNKS_PER_ROW
    p0 = (wid % _CHUNKS_PER_ROW) * _CHUNK

    bytes_v[pl.ds(0, _L)] = jnp.zeros((_L,), jnp.int32)
    pltpu.sync_copy(byte_hbm.at[pl.ds(b * _T, _T)], bytes_v.at[pl.ds(_PAD, _T)])

    for g in range(_CHUNK // _L):
        t0 = p0 + g * _L
        v0 = bytes_v[pl.ds(_PAD + t0, _L)]
        v1 = bytes_v[pl.ds(_PAD + t0 - 1, _L)]
        v2 = bytes_v[pl.ds(_PAD + t0 - 2, _L)]
        v3 = bytes_v[pl.ds(_PAD + t0 - 3, _L)]
        h2 = v1 * _PRIME + v0
        h3 = v2 * (_PRIME * _PRIME) + h2
        h4 = lax.rem(v3 * (_PRIME * _PRIME * _PRIME) + h3, _VOCAB)
        idx2_v[pl.ds(g * _L, _L)] = h2
        idx3_v[pl.ds(g * _L, _L)] = h3
        idx4_v[pl.ds(g * _L, _L)] = h4

    c2 = pltpu.async_copy(emb2_hbm.at[idx2_v], e2_v, sem)
    c3 = pltpu.async_copy(emb3_hbm.at[idx3_v], e3_v, sem)
    c4 = pltpu.async_copy(emb4_hbm.at[idx4_v], e4_v, sem)
    c2.wait()
    c3.wait()
    c4.wait()

    def acc_body(i, _):
        for c in range(_DIM // _L):
            sl = pl.ds(c * _L, _L)
            s = e2_v[i, sl] + e3_v[i, sl] + e4_v[i, sl]
            out_v[i, sl] = s * 0.25
        return 0

    lax.fori_loop(0, _CHUNK, acc_body, 0)

    @pl.when(p0 == 0)
    def _fixup():
        for c in range(_DIM // _L):
            sl = pl.ds(c * _L, _L)
            out_v[0, sl] = jnp.zeros((_L,), jnp.float32)
            out_v[1, sl] = e2_v[1, sl] * 0.25
            out_v[2, sl] = (e2_v[2, sl] + e3_v[2, sl]) * 0.25

    pltpu.sync_copy(out_v, out_hbm.at[b, pl.ds(p0, _CHUNK)])


@jax.jit
def kernel(byte_ids, emb_2, emb_3, emb_4):
    mesh = plsc.VectorSubcoreMesh(core_axis_name="c", subcore_axis_name="s")
    f = functools.partial(
        pl.kernel,
        mesh=mesh,
        compiler_params=pltpu.CompilerParams(use_tc_tiling_on_sc=False),
        out_type=jax.ShapeDtypeStruct((_B, _T, _DIM), jnp.float32),
        scratch_types=[
            pltpu.VMEM((_PAD + _T,), jnp.int32),
            pltpu.VMEM((_CHUNK,), jnp.int32),
            pltpu.VMEM((_CHUNK,), jnp.int32),
            pltpu.VMEM((_CHUNK,), jnp.int32),
            pltpu.VMEM((_CHUNK, _DIM), jnp.float32),
            pltpu.VMEM((_CHUNK, _DIM), jnp.float32),
            pltpu.VMEM((_CHUNK, _DIM), jnp.float32),
            pltpu.VMEM((_CHUNK, _DIM), jnp.float32),
            pltpu.SemaphoreType.DMA,
        ],
    )(_body)
    return f(byte_ids.reshape(-1), emb_2, emb_3, emb_4)
